# baseline (speedup 1.0000x reference)
import jax
import jax.numpy as jnp
from jax import lax
from jax.experimental import pallas as pl
from jax.experimental.pallas import tpu as pltpu
from jax.experimental.pallas import tpu_sc as plsc

N = 10000
E = 320000
D = 128
NC = 2
NS = 16
NW = NC * NS
B = 128
CH = 80
EPW = CH * B
EP = EPW * NW
NP = 10112
RPS = NP // NS
LANES = 16
DV = D // LANES


def _matmul_body(x_ref, w_ref, o_ref):
    o_ref[...] = jnp.dot(x_ref[...], w_ref[...], preferred_element_type=jnp.float32)


def _combine_body(p_ref, b_ref, o_ref):
    o_ref[...] = p_ref[0] + p_ref[1] + b_ref[...]


def _sc_body(support_hbm, spack_hbm, wpack_hbm, out_hbm,
             spackv, wpackv, srcb, dstb, wb, rows0, rows1, acc,
             sem0, sem1):
    c = lax.axis_index("c")
    s = lax.axis_index("s")
    wid = c * NS + s

    pltpu.sync_copy(spack_hbm.at[wid], spackv)
    pltpu.sync_copy(wpack_hbm.at[wid], wpackv)

    zeros16 = jnp.zeros((LANES,), jnp.float32)

    def zero_row(r, carry):
        for d in range(DV):
            rows0[r, pl.ds(d * LANES, LANES)] = zeros16
        return carry

    lax.fori_loop(0, B, zero_row, 0)
    for j in range(RPS // B):
        pltpu.sync_copy(rows0, acc.at[pl.ds(s * RPS + j * B, B)])
    pltpu.sync_copy(rows0.at[pl.ds(0, RPS % B)],
                    acc.at[pl.ds(s * RPS + (RPS // B) * B, RPS % B)])
    plsc.subcore_barrier()

    mask16 = jnp.full((LANES,), 0xFFFF, jnp.int32)
    hi16 = jnp.full((LANES,), 16, jnp.int32)

    def unpack(k, b):
        def idx16(eb, carry):
            sl = pl.ds(eb * LANES, LANES)
            p = spackv[k, sl]
            srcb[b, sl] = lax.bitwise_and(p, mask16)
            dstb[b, sl] = lax.shift_right_logical(p, hi16)
            return carry

        lax.fori_loop(0, B // LANES, idx16, 0)

        def w32(g, carry):
            p = wpackv[k // 2, pl.ds(b * (B // 2) + g * LANES, LANES)]
            lo = lax.bitcast_convert_type(
                lax.shift_left(p, hi16), jnp.float32)
            hi = lax.bitcast_convert_type(
                lax.bitwise_and(p, jnp.full((LANES,), -65536, jnp.int32)),
                jnp.float32)
            wb[b, pl.ds(2 * g * LANES, LANES)] = lo
            wb[b, pl.ds((2 * g + 1) * LANES, LANES)] = hi
            return carry

        lax.fori_loop(0, B // (2 * LANES), w32, 0)

    def scale(b, rb):
        def edge16(eb, c2):
            w16 = wb[b, pl.ds(eb * LANES, LANES)]
            for j in range(LANES):
                wvec = lax.gather(
                    w16, jnp.full((LANES, 1), j, jnp.int32),
                    dimension_numbers=lax.GatherDimensionNumbers(
                        offset_dims=(), collapsed_slice_dims=(0,),
                        start_index_map=(0,)),
                    slice_sizes=(1,),
                    mode=lax.GatherScatterMode.PROMISE_IN_BOUNDS)
                e = eb * LANES + j
                for d in range(DV):
                    sl = pl.ds(d * LANES, LANES)
                    rb[e, sl] = rb[e, sl] * wvec
            return c2

        lax.fori_loop(0, B // LANES, edge16, 0)

    rows_sem = ((rows0, sem0), (rows1, sem1))

    def step(k, b):
        rb, sb = rows_sem[b]
        pb = 1 - b
        prb = rows_sem[pb][0]
        unpack(k, b)
        cp = pltpu.async_copy(support_hbm.at[srcb.at[b]], rb, sb)
        scale(pb, prb)
        cp.wait()
        pltpu.sync_copy(prb, acc.at[dstb.at[pb]], add=True)

    unpack(0, 0)
    pltpu.async_copy(support_hbm.at[srcb.at[0]], rows0, sem0).wait()
    step(1, 1)

    def pair(jj, carry):
        k = jj * 2 + 2
        step(k, 0)
        step(k + 1, 1)
        return carry

    lax.fori_loop(0, (CH - 2) // 2, pair, 0)
    scale(1, rows1)
    pltpu.sync_copy(rows1, acc.at[dstb.at[1]], add=True)
    plsc.subcore_barrier()

    pltpu.sync_copy(acc.at[pl.ds(s * RPS, RPS)],
                    out_hbm.at[c, pl.ds(s * RPS, RPS)])


_sc_call = pl.kernel(
    _sc_body,
    out_type=jax.ShapeDtypeStruct((NC, NP, D), jnp.float32),
    mesh=plsc.VectorSubcoreMesh(core_axis_name="c", subcore_axis_name="s"),
    scratch_types=[
        pltpu.VMEM((CH, B), jnp.int32),
        pltpu.VMEM((CH // 2, B), jnp.int32),
        pltpu.VMEM((2, B), jnp.int32),
        pltpu.VMEM((2, B), jnp.int32),
        pltpu.VMEM((2, B), jnp.float32),
        pltpu.VMEM((B, D), jnp.float32),
        pltpu.VMEM((B, D), jnp.float32),
        pltpu.VMEM_SHARED((NP, D), jnp.float32),
        pltpu.SemaphoreType.DMA,
        pltpu.SemaphoreType.DMA,
    ],
)


def kernel(input, adj_edge_index, adj_edge_weight, W, b):
    support = pl.pallas_call(
        _matmul_body,
        out_shape=jax.ShapeDtypeStruct((N, D), jnp.float32),
        grid=(10,),
        in_specs=[pl.BlockSpec((N // 10, D), lambda i: (i, 0)),
                  pl.BlockSpec((D, D), lambda i: (0, 0))],
        out_specs=pl.BlockSpec((N // 10, D), lambda i: (i, 0)),
    )(input, W)

    pad = EP - E
    src = jnp.pad(adj_edge_index[0], (0, pad))
    dst = jnp.pad(adj_edge_index[1], (0, pad))
    spack = (src | (dst << 16)).reshape(NW, CH, B)
    wb16 = jnp.pad(adj_edge_weight, (0, pad)).astype(jnp.bfloat16)
    wpair = wb16.reshape(-1, 2, LANES).transpose(0, 2, 1)
    wpack = lax.bitcast_convert_type(wpair, jnp.int32).reshape(NW, CH // 2, B)

    partials = _sc_call(support, spack, wpack)

    out = pl.pallas_call(
        _combine_body,
        out_shape=jax.ShapeDtypeStruct((N, D), jnp.float32),
        grid=(10,),
        in_specs=[pl.BlockSpec((NC, N // 10, D), lambda i: (0, i, 0)),
                  pl.BlockSpec((1, D), lambda i: (0, 0))],
        out_specs=pl.BlockSpec((N // 10, D), lambda i: (i, 0)),
    )(partials, b.reshape(1, D))
    return out


# speedup vs baseline: 1.5129x; 1.5129x over previous
import jax
import jax.numpy as jnp
from jax import lax
from jax.experimental import pallas as pl
from jax.experimental.pallas import tpu as pltpu
from jax.experimental.pallas import tpu_sc as plsc

N = 10000
E = 320000
D = 128
NC = 2
NS = 16
NW = NC * NS
B = 128
CH = 79
EPW = CH * B
EP = EPW * NW
NP = 10240
RPS = NP // NS
LANES = 16
DV = D // LANES


def _matmul_body(x_ref, w_ref, o_ref):
    o_ref[...] = jnp.dot(x_ref[...], w_ref[...], preferred_element_type=jnp.float32)


def _combine_body(p_ref, b_ref, o_ref):
    o_ref[...] = p_ref[0] + p_ref[1] + b_ref[...]


def _sc_body(support_hbm, src_hbm, dst_hbm, w_hbm, out_hbm,
             srcv, dstv, wv, rows, acc, sem):
    c = lax.axis_index("c")
    s = lax.axis_index("s")
    wid = c * NS + s

    pltpu.sync_copy(src_hbm.at[wid], srcv)
    pltpu.sync_copy(dst_hbm.at[wid], dstv)
    pltpu.sync_copy(w_hbm.at[wid], wv)

    zeros16 = jnp.zeros((LANES,), jnp.float32)

    def zero_row(r, carry):
        for d in range(DV):
            rows[r, pl.ds(d * LANES, LANES)] = zeros16
        return carry

    lax.fori_loop(0, B, zero_row, 0)
    for j in range(RPS // B):
        pltpu.sync_copy(rows, acc.at[pl.ds(s * RPS + j * B, B)])
    plsc.subcore_barrier()

    def chunk(k, carry):
        pltpu.async_copy(support_hbm.at[srcv.at[k]], rows, sem).wait()

        def edge16(eb, c2):
            w16 = wv[pl.ds(k * B + eb * LANES, LANES)]
            for j in range(LANES):
                wvec = lax.gather(
                    w16, jnp.full((LANES, 1), j, jnp.int32),
                    dimension_numbers=lax.GatherDimensionNumbers(
                        offset_dims=(), collapsed_slice_dims=(0,),
                        start_index_map=(0,)),
                    slice_sizes=(1,),
                    mode=lax.GatherScatterMode.PROMISE_IN_BOUNDS)
                e = eb * LANES + j
                for d in range(DV):
                    sl = pl.ds(d * LANES, LANES)
                    rows[e, sl] = rows[e, sl] * wvec
            return c2

        lax.fori_loop(0, B // LANES, edge16, 0)

        pltpu.sync_copy(rows, acc.at[dstv.at[k]], add=True)
        return carry

    lax.fori_loop(0, CH, chunk, 0)
    plsc.subcore_barrier()

    pltpu.sync_copy(acc.at[pl.ds(s * RPS, RPS)],
                    out_hbm.at[c, pl.ds(s * RPS, RPS)])


_sc_call = pl.kernel(
    _sc_body,
    out_type=jax.ShapeDtypeStruct((NC, NP, D), jnp.float32),
    mesh=plsc.VectorSubcoreMesh(core_axis_name="c", subcore_axis_name="s"),
    compiler_params=pltpu.CompilerParams(use_tc_tiling_on_sc=False),
    scratch_types=[
        pltpu.VMEM((CH, B), jnp.int32),
        pltpu.VMEM((CH, B), jnp.int32),
        pltpu.VMEM((EPW,), jnp.float32),
        pltpu.VMEM((B, D), jnp.float32),
        pltpu.VMEM_SHARED((NP, D), jnp.float32),
        pltpu.SemaphoreType.DMA,
    ],
)


def kernel(input, adj_edge_index, adj_edge_weight, W, b):
    support = pl.pallas_call(
        _matmul_body,
        out_shape=jax.ShapeDtypeStruct((N, D), jnp.float32),
        grid=(10,),
        in_specs=[pl.BlockSpec((N // 10, D), lambda i: (i, 0)),
                  pl.BlockSpec((D, D), lambda i: (0, 0))],
        out_specs=pl.BlockSpec((N // 10, D), lambda i: (i, 0)),
    )(input, W)

    pad = EP - E
    src = jnp.pad(adj_edge_index[0], (0, pad)).reshape(NW, CH, B)
    dst = jnp.pad(adj_edge_index[1], (0, pad)).reshape(NW, CH, B)
    w = jnp.pad(adj_edge_weight, (0, pad)).reshape(NW, EPW)

    partials = _sc_call(support, src, dst, w)

    out = pl.pallas_call(
        _combine_body,
        out_shape=jax.ShapeDtypeStruct((N, D), jnp.float32),
        grid=(10,),
        in_specs=[pl.BlockSpec((NC, N // 10, D), lambda i: (0, i, 0)),
                  pl.BlockSpec((1, D), lambda i: (0, 0))],
        out_specs=pl.BlockSpec((N // 10, D), lambda i: (i, 0)),
    )(partials, b.reshape(1, D))
    return out
